# Initial kernel scaffold; baseline (speedup 1.0000x reference)
import jax
import jax.numpy as jnp
from jax.experimental import pallas as pl


def kernel(oldx, W_in, b_in, W_out):
    raise NotImplementedError("write your pallas kernel here")


# speedup vs baseline: 1.2171x; 1.2171x over previous
import jax
import jax.numpy as jnp
from jax.experimental import pallas as pl

DIM = 2048
DEPTH = 7
PAR = 16
N_NODES = 2 ** (DEPTH + 1) - 1
NPAD = N_NODES + 1

_DN_T = (((1,), (1,)), ((), ()))
_DN = (((1,), (0,)), ((), ()))


def _fff_block(xh_ref, xl_ref, wh_ref, wl_ref, b_in_ref, w_out_ref, o_ref):
    xh = xh_ref[...]
    bt = xh.shape[0]
    logits = jax.lax.dot_general(xh, wh_ref[0], _DN_T,
                                 preferred_element_type=jnp.float32)
    logits += jax.lax.dot_general(xh, wl_ref[0], _DN_T,
                                  preferred_element_type=jnp.float32)
    logits += jax.lax.dot_general(xl_ref[...], wh_ref[0], _DN_T,
                                  preferred_element_type=jnp.float32)
    logits += b_in_ref[0]
    dec = logits > 0.0
    act = logits * jax.nn.sigmoid(logits)
    iota = jax.lax.broadcasted_iota(jnp.int32, (bt, NPAD), 1)
    cur = jnp.zeros((bt, 1), jnp.int32)
    oh = iota == cur
    mask = oh.astype(jnp.float32)
    for d in range(DEPTH):
        move = jnp.sum(jnp.logical_and(oh, dec).astype(jnp.int32),
                       axis=1, keepdims=True)
        cur = (cur - (2 ** d - 1)) * 2 + move + (2 ** (d + 1) - 1)
        oh = iota == cur
        mask = mask + oh.astype(jnp.float32)
    act_m = (act * mask).astype(jnp.bfloat16)
    contrib = jax.lax.dot_general(act_m, w_out_ref[0], _DN,
                                  preferred_element_type=jnp.float32)

    @pl.when(pl.program_id(1) == 0)
    def _():
        o_ref[...] = contrib

    @pl.when(pl.program_id(1) != 0)
    def _():
        o_ref[...] = o_ref[...] + contrib


def kernel(oldx, W_in, b_in, W_out):
    x = oldx.reshape(-1, DIM)
    B = x.shape[0]
    x_hi = x.astype(jnp.bfloat16)
    x_lo = (x - x_hi.astype(jnp.float32)).astype(jnp.bfloat16)
    w_in3 = jnp.pad(W_in.reshape(PAR, N_NODES, DIM), ((0, 0), (0, 1), (0, 0)))
    w_hi = w_in3.astype(jnp.bfloat16)
    w_lo = (w_in3 - w_hi.astype(jnp.float32)).astype(jnp.bfloat16)
    b3 = jnp.pad(b_in.reshape(PAR, 1, N_NODES), ((0, 0), (0, 0), (0, 1)))
    w_out3 = jnp.pad(W_out.T.reshape(PAR, N_NODES, DIM),
                     ((0, 0), (0, 1), (0, 0))).astype(jnp.bfloat16)
    bt = 256 if B % 256 == 0 else B
    out = pl.pallas_call(
        _fff_block,
        grid=(B // bt, PAR),
        in_specs=[
            pl.BlockSpec((bt, DIM), lambda i, p: (i, 0)),
            pl.BlockSpec((bt, DIM), lambda i, p: (i, 0)),
            pl.BlockSpec((1, NPAD, DIM), lambda i, p: (p, 0, 0)),
            pl.BlockSpec((1, NPAD, DIM), lambda i, p: (p, 0, 0)),
            pl.BlockSpec((1, 1, NPAD), lambda i, p: (p, 0, 0)),
            pl.BlockSpec((1, NPAD, DIM), lambda i, p: (p, 0, 0)),
        ],
        out_specs=pl.BlockSpec((bt, DIM), lambda i, p: (i, 0)),
        out_shape=jax.ShapeDtypeStruct((B, DIM), jnp.float32),
    )(x_hi, x_lo, w_hi, w_lo, b3, w_out3)
    return out.reshape(oldx.shape)


# speedup vs baseline: 2.1046x; 1.7292x over previous
import numpy as np
import jax
import jax.numpy as jnp
from jax.experimental import pallas as pl

DIM = 2048
DEPTH = 7
PAR = 16
N_NODES = 2 ** (DEPTH + 1) - 1
NPAD = N_NODES + 1
UNROLL = 2

_DN_T = (((1,), (1,)), ((), ()))


def _ancestor_tables():
    anc = np.zeros((NPAD, NPAD), np.float32)
    depth = np.full((1, NPAD), -1.0, np.float32)
    for n in range(N_NODES):
        d = 0
        m = n
        while m != 0:
            parent = (m - 1) // 2
            anc[n, parent] = 1.0 if (m - 1) % 2 else -1.0
            m = parent
            d += 1
        depth[0, n] = d
    return anc, depth


_ANC, _DEPTH_OF = _ancestor_tables()


def _fff_block(xh_ref, xl_ref, wh_ref, wl_ref, b_in_ref, anc_ref, dep_ref,
               w_out_ref, o_ref):
    xh = xh_ref[...]
    bt = xh.shape[0]
    wh = wh_ref[...].reshape(UNROLL * NPAD, DIM)
    wl = wl_ref[...].reshape(UNROLL * NPAD, DIM)
    logits = jax.lax.dot_general(xh, wh, _DN_T,
                                 preferred_element_type=jnp.float32)
    logits += jax.lax.dot_general(xh, wl, _DN_T,
                                  preferred_element_type=jnp.float32)
    logits += jax.lax.dot_general(xl_ref[...], wh, _DN_T,
                                  preferred_element_type=jnp.float32)
    logits += b_in_ref[0]
    s = jnp.where(logits > 0.0, 1.0, -1.0).astype(jnp.bfloat16)
    anc = anc_ref[...]
    score = jnp.concatenate(
        [jax.lax.dot_general(s[:, u * NPAD:(u + 1) * NPAD], anc, _DN_T,
                             preferred_element_type=jnp.float32)
         for u in range(UNROLL)], axis=1)
    act = logits * jax.nn.sigmoid(logits)
    act_m = jnp.where(score == dep_ref[...], act, 0.0).astype(jnp.bfloat16)
    contrib = jax.lax.dot_general(
        act_m, w_out_ref[...].reshape(UNROLL * NPAD, DIM),
        (((1,), (0,)), ((), ())),
        preferred_element_type=jnp.float32)

    @pl.when(pl.program_id(1) == 0)
    def _():
        o_ref[...] = contrib

    @pl.when(pl.program_id(1) != 0)
    def _():
        o_ref[...] = o_ref[...] + contrib


def kernel(oldx, W_in, b_in, W_out):
    x = oldx.reshape(-1, DIM)
    B = x.shape[0]
    x_hi = x.astype(jnp.bfloat16)
    x_lo = (x - x_hi.astype(jnp.float32)).astype(jnp.bfloat16)
    w_in3 = jnp.pad(W_in.reshape(PAR, N_NODES, DIM), ((0, 0), (0, 1), (0, 0)))
    w_hi = w_in3.astype(jnp.bfloat16)
    w_lo = (w_in3 - w_hi.astype(jnp.float32)).astype(jnp.bfloat16)
    b3 = jnp.pad(b_in.reshape(PAR, 1, N_NODES), ((0, 0), (0, 0), (0, 1)))
    b3 = b3.reshape(PAR // UNROLL, 1, UNROLL * NPAD)
    anc = jnp.asarray(_ANC).astype(jnp.bfloat16)
    dep = jnp.asarray(np.tile(_DEPTH_OF, (1, UNROLL)))
    w_out3 = jnp.pad(W_out.T.reshape(PAR, N_NODES, DIM),
                     ((0, 0), (0, 1), (0, 0))).astype(jnp.bfloat16)
    bt = 256 if B % 256 == 0 else B
    g2 = PAR // UNROLL
    out = pl.pallas_call(
        _fff_block,
        grid=(B // bt, g2),
        in_specs=[
            pl.BlockSpec((bt, DIM), lambda i, p: (i, 0)),
            pl.BlockSpec((bt, DIM), lambda i, p: (i, 0)),
            pl.BlockSpec((UNROLL, NPAD, DIM), lambda i, p: (p, 0, 0)),
            pl.BlockSpec((UNROLL, NPAD, DIM), lambda i, p: (p, 0, 0)),
            pl.BlockSpec((1, 1, UNROLL * NPAD), lambda i, p: (p, 0, 0)),
            pl.BlockSpec((NPAD, NPAD), lambda i, p: (0, 0)),
            pl.BlockSpec((1, UNROLL * NPAD), lambda i, p: (0, 0)),
            pl.BlockSpec((UNROLL, NPAD, DIM), lambda i, p: (p, 0, 0)),
        ],
        out_specs=pl.BlockSpec((bt, DIM), lambda i, p: (i, 0)),
        out_shape=jax.ShapeDtypeStruct((B, DIM), jnp.float32),
    )(x_hi, x_lo, w_hi, w_lo, b3, anc, dep, w_out3)
    return out.reshape(oldx.shape)


# speedup vs baseline: 2.5028x; 1.1892x over previous
import numpy as np
import jax
import jax.numpy as jnp
from jax.experimental import pallas as pl

DIM = 2048
DEPTH = 7
PAR = 16
N_NODES = 2 ** (DEPTH + 1) - 1
NPAD = N_NODES + 1
UNROLL = 4

_DN_T = (((1,), (1,)), ((), ()))


def _ancestor_tables():
    anc = np.zeros((NPAD, NPAD), np.float32)
    depth = np.full((1, NPAD), -1.0, np.float32)
    for n in range(N_NODES):
        d = 0
        m = n
        while m != 0:
            parent = (m - 1) // 2
            anc[n, parent] = 1.0 if (m - 1) % 2 else -1.0
            m = parent
            d += 1
        depth[0, n] = d
    return anc, depth


_ANC, _DEPTH_OF = _ancestor_tables()


def _fff_block(xh_ref, xl_ref, wh_ref, wl_ref, b_in_ref, anc_ref, dep_ref,
               w_out_ref, o_ref):
    xh = xh_ref[...]
    bt = xh.shape[0]
    wh = wh_ref[...].reshape(UNROLL * NPAD, DIM)
    wl = wl_ref[...].reshape(UNROLL * NPAD, DIM)
    logits = jax.lax.dot_general(xh, wh, _DN_T,
                                 preferred_element_type=jnp.float32)
    logits += jax.lax.dot_general(xh, wl, _DN_T,
                                  preferred_element_type=jnp.float32)
    logits += jax.lax.dot_general(xl_ref[...], wh, _DN_T,
                                  preferred_element_type=jnp.float32)
    logits += b_in_ref[0]
    s = jnp.where(logits > 0.0, 1.0, -1.0).astype(jnp.bfloat16)
    anc = anc_ref[...]
    score = jnp.concatenate(
        [jax.lax.dot_general(s[:, u * NPAD:(u + 1) * NPAD], anc, _DN_T,
                             preferred_element_type=jnp.float32)
         for u in range(UNROLL)], axis=1)
    act = logits * jax.nn.sigmoid(logits)
    act_m = jnp.where(score == dep_ref[...], act, 0.0).astype(jnp.bfloat16)
    contrib = jax.lax.dot_general(
        act_m, w_out_ref[...].reshape(UNROLL * NPAD, DIM),
        (((1,), (0,)), ((), ())),
        preferred_element_type=jnp.float32)

    @pl.when(pl.program_id(1) == 0)
    def _():
        o_ref[...] = contrib

    @pl.when(pl.program_id(1) != 0)
    def _():
        o_ref[...] = o_ref[...] + contrib


def kernel(oldx, W_in, b_in, W_out):
    x = oldx.reshape(-1, DIM)
    B = x.shape[0]
    x_hi = x.astype(jnp.bfloat16)
    x_lo = (x - x_hi.astype(jnp.float32)).astype(jnp.bfloat16)
    w_in3 = jnp.pad(W_in.reshape(PAR, N_NODES, DIM), ((0, 0), (0, 1), (0, 0)))
    w_hi = w_in3.astype(jnp.bfloat16)
    w_lo = (w_in3 - w_hi.astype(jnp.float32)).astype(jnp.bfloat16)
    b3 = jnp.pad(b_in.reshape(PAR, 1, N_NODES), ((0, 0), (0, 0), (0, 1)))
    b3 = b3.reshape(PAR // UNROLL, 1, UNROLL * NPAD)
    anc = jnp.asarray(_ANC).astype(jnp.bfloat16)
    dep = jnp.asarray(np.tile(_DEPTH_OF, (1, UNROLL)))
    w_out3 = jnp.pad(W_out.T.reshape(PAR, N_NODES, DIM),
                     ((0, 0), (0, 1), (0, 0))).astype(jnp.bfloat16)
    bt = 512 if B % 512 == 0 else B
    g2 = PAR // UNROLL
    out = pl.pallas_call(
        _fff_block,
        grid=(B // bt, g2),
        in_specs=[
            pl.BlockSpec((bt, DIM), lambda i, p: (i, 0)),
            pl.BlockSpec((bt, DIM), lambda i, p: (i, 0)),
            pl.BlockSpec((UNROLL, NPAD, DIM), lambda i, p: (p, 0, 0)),
            pl.BlockSpec((UNROLL, NPAD, DIM), lambda i, p: (p, 0, 0)),
            pl.BlockSpec((1, 1, UNROLL * NPAD), lambda i, p: (p, 0, 0)),
            pl.BlockSpec((NPAD, NPAD), lambda i, p: (0, 0)),
            pl.BlockSpec((1, UNROLL * NPAD), lambda i, p: (0, 0)),
            pl.BlockSpec((UNROLL, NPAD, DIM), lambda i, p: (p, 0, 0)),
        ],
        out_specs=pl.BlockSpec((bt, DIM), lambda i, p: (i, 0)),
        out_shape=jax.ShapeDtypeStruct((B, DIM), jnp.float32),
    )(x_hi, x_lo, w_hi, w_lo, b3, anc, dep, w_out3)
    return out.reshape(oldx.shape)


# speedup vs baseline: 2.6320x; 1.0516x over previous
import numpy as np
import jax
import jax.numpy as jnp
from jax.experimental import pallas as pl

DIM = 2048
DEPTH = 7
PAR = 16
N_NODES = 2 ** (DEPTH + 1) - 1
N_INT = 2 ** DEPTH - 1
HALF = 128
NPAD = 256
UNROLL = 4
GW = UNROLL * HALF

_DN_T = (((1,), (1,)), ((), ()))


def _tables():
    pos = np.array([n if n < N_INT else n + 1 for n in range(N_NODES)])
    anc = np.zeros((NPAD, HALF), np.float32)
    depth = np.full((NPAD,), -1.0, np.float32)
    for n in range(N_NODES):
        d = 0
        m = n
        while m != 0:
            parent = (m - 1) // 2
            anc[pos[n], parent] = 1.0 if (m - 1) % 2 else -1.0
            m = parent
            d += 1
        depth[pos[n]] = d
    anc_bd = np.zeros((2 * GW, GW), np.float32)
    dep_bd = np.zeros((1, 2 * GW), np.float32)
    for u in range(UNROLL):
        anc_bd[u * HALF:(u + 1) * HALF, u * HALF:(u + 1) * HALF] = \
            anc[:HALF]
        anc_bd[GW + u * HALF:GW + (u + 1) * HALF,
               u * HALF:(u + 1) * HALF] = anc[HALF:]
        dep_bd[0, u * HALF:(u + 1) * HALF] = depth[:HALF]
        dep_bd[0, GW + u * HALF:GW + (u + 1) * HALF] = depth[HALF:]
    return anc_bd, dep_bd


_ANC_BD, _DEP_BD = _tables()


def _split_halves(w3):
    g = PAR // UNROLL
    w3 = jnp.concatenate([
        w3[:, :N_INT], jnp.zeros_like(w3[:, :1]), w3[:, N_INT:]], axis=1)
    w4 = w3.reshape(g, UNROLL, NPAD, -1)
    return jnp.concatenate(
        [w4[:, :, :HALF].reshape(g, GW, -1),
         w4[:, :, HALF:].reshape(g, GW, -1)], axis=1)


def _fff_block(xh_ref, xl_ref, wh_ref, wl_ref, bi_ref, bl_ref, anc_ref,
               dep_ref, w_out_ref, o_ref):
    xh = xh_ref[...]
    xl = xl_ref[...]
    w_int_h = wh_ref[0, :GW, :]
    w_leaf_h = wh_ref[0, GW:, :]
    wl = wl_ref[0]
    li = jax.lax.dot_general(xh, w_int_h, _DN_T,
                             preferred_element_type=jnp.float32)
    li += jax.lax.dot_general(xh, wl, _DN_T,
                              preferred_element_type=jnp.float32)
    li += jax.lax.dot_general(xl, w_int_h, _DN_T,
                              preferred_element_type=jnp.float32)
    li += bi_ref[0]
    ll = jax.lax.dot_general(xh, w_leaf_h, _DN_T,
                             preferred_element_type=jnp.float32)
    ll += bl_ref[0]
    s = jnp.where(li > 0.0, 1.0, -1.0).astype(jnp.bfloat16)
    score = jax.lax.dot_general(s, anc_ref[...], _DN_T,
                                preferred_element_type=jnp.float32)
    act_i = li * jax.nn.sigmoid(li)
    act_l = ll * jax.nn.sigmoid(ll)
    act = jnp.concatenate([act_i, act_l], axis=1)
    act_m = jnp.where(score == dep_ref[...], act, 0.0).astype(jnp.bfloat16)
    contrib = jax.lax.dot_general(act_m, w_out_ref[0],
                                  (((1,), (0,)), ((), ())),
                                  preferred_element_type=jnp.float32)

    @pl.when(pl.program_id(1) == 0)
    def _():
        o_ref[...] = contrib

    @pl.when(pl.program_id(1) != 0)
    def _():
        o_ref[...] = o_ref[...] + contrib


def kernel(oldx, W_in, b_in, W_out):
    x = oldx.reshape(-1, DIM)
    B = x.shape[0]
    g = PAR // UNROLL
    x_hi = x.astype(jnp.bfloat16)
    x_lo = (x - x_hi.astype(jnp.float32)).astype(jnp.bfloat16)
    w_in_s = _split_halves(W_in.reshape(PAR, N_NODES, DIM))
    w_hi = w_in_s.astype(jnp.bfloat16)
    w_lo = (w_in_s[:, :GW] - w_hi[:, :GW].astype(jnp.float32)
            ).astype(jnp.bfloat16)
    b_s = _split_halves(b_in.reshape(PAR, N_NODES, 1))[..., 0]
    b_int = b_s[:, None, :GW]
    b_leaf = b_s[:, None, GW:]
    anc = jnp.asarray(_ANC_BD).astype(jnp.bfloat16)
    dep = jnp.asarray(_DEP_BD)
    w_out_s = _split_halves(W_out.T.reshape(PAR, N_NODES, DIM)
                            ).astype(jnp.bfloat16)
    bt = 512 if B % 512 == 0 else B
    out = pl.pallas_call(
        _fff_block,
        grid=(B // bt, g),
        in_specs=[
            pl.BlockSpec((bt, DIM), lambda i, p: (i, 0)),
            pl.BlockSpec((bt, DIM), lambda i, p: (i, 0)),
            pl.BlockSpec((1, 2 * GW, DIM), lambda i, p: (p, 0, 0)),
            pl.BlockSpec((1, GW, DIM), lambda i, p: (p, 0, 0)),
            pl.BlockSpec((1, 1, GW), lambda i, p: (p, 0, 0)),
            pl.BlockSpec((1, 1, GW), lambda i, p: (p, 0, 0)),
            pl.BlockSpec((2 * GW, GW), lambda i, p: (0, 0)),
            pl.BlockSpec((1, 2 * GW), lambda i, p: (0, 0)),
            pl.BlockSpec((1, 2 * GW, DIM), lambda i, p: (p, 0, 0)),
        ],
        out_specs=pl.BlockSpec((bt, DIM), lambda i, p: (i, 0)),
        out_shape=jax.ShapeDtypeStruct((B, DIM), jnp.float32),
    )(x_hi, x_lo, w_hi, w_lo, b_int, b_leaf, anc, dep, w_out_s)
    return out.reshape(oldx.shape)
